# Initial kernel scaffold; baseline (speedup 1.0000x reference)
import jax
import jax.numpy as jnp
from jax.experimental import pallas as pl


def kernel(knowledge_id, category, difficulty, tags, title_tokens, title_lengths, learner_count, rating, duration, days_since_publish, knowledge_table, category_table, difficulty_table, tag_table, learner_table, rating_table, duration_table, freshness_table, title_emb, W_ih_f, W_hh_f, b_ih_f, b_hh_f, W_ih_b, W_hh_b, b_ih_b, b_hh_b, W1, b1, g1, be1, W2, b2, g2, be2):
    raise NotImplementedError("write your pallas kernel here")


# speedup vs baseline: 1.9952x; 1.9952x over previous
import functools

import numpy as np
import jax
import jax.numpy as jnp
from jax import lax
from jax.experimental import pallas as pl
from jax.experimental.pallas import tpu as pltpu
from jax.experimental.pallas import tpu_sc as plsc

B = 4096
ED = 128
TAGS = 10
TL = 50
HID = 64
TD = 64
FIN = ED * 8 + 2 * HID

NC = 2
NS = 16
NW = NC * NS
PW = B // NW
CH = 128

_f32 = jnp.float32

BT = 256
NB = B // BT


def _sc_gather(kid, cat, dif, tags_flat, ttok_t, ktab, ctab, dtab, tgtab, etab):
    mesh = plsc.VectorSubcoreMesh(core_axis_name="c", subcore_axis_name="s")

    @functools.partial(
        pl.kernel,
        out_type=[
            jax.ShapeDtypeStruct((B, ED), _f32),
            jax.ShapeDtypeStruct((B, ED), _f32),
            jax.ShapeDtypeStruct((B, ED), _f32),
            jax.ShapeDtypeStruct((B * TAGS, ED), _f32),
            jax.ShapeDtypeStruct((TL * B, TD), _f32),
        ],
        mesh=mesh,
        compiler_params=pltpu.CompilerParams(use_tc_tiling_on_sc=False),
        scratch_types=[
            pltpu.VMEM((CH,), jnp.int32),
            pltpu.VMEM((CH, ED), _f32),
            pltpu.VMEM((CH, TD), _f32),
            pltpu.SemaphoreType.DMA,
        ],
    )
    def k(kid_h, cat_h, dif_h, tags_h, ttok_h, ktab_h, ctab_h, dtab_h,
          tgtab_h, etab_h, ko_h, co_h, do_h, tgo_h, tto_h,
          idx_v, rows_v, rows64_v, sem):
        cid = lax.axis_index("c")
        sid = lax.axis_index("s")
        wid = sid * NC + cid
        base = wid * PW

        def gather128(src_idx_h, table_h, out_h, off):
            pltpu.sync_copy(src_idx_h.at[pl.ds(off, CH)], idx_v)
            pltpu.async_copy(table_h.at[idx_v], rows_v, sem).wait()
            pltpu.sync_copy(rows_v, out_h.at[pl.ds(off, CH)])

        gather128(kid_h, ktab_h, ko_h, base)
        gather128(cat_h, ctab_h, co_h, base)
        gather128(dif_h, dtab_h, do_h, base)

        def tag_body(j, carry):
            gather128(tags_h, tgtab_h, tgo_h, base * TAGS + j * CH)
            return carry
        lax.fori_loop(0, PW * TAGS // CH, tag_body, 0)

        tbase = wid * (TL * B // NW)

        def ttl_body(j, carry):
            off = tbase + j * CH
            pltpu.sync_copy(ttok_h.at[pl.ds(off, CH)], idx_v)
            pltpu.async_copy(etab_h.at[idx_v], rows64_v, sem).wait()
            pltpu.sync_copy(rows64_v, tto_h.at[pl.ds(off, CH)])
            return carry
        lax.fori_loop(0, TL * B // NW // CH, ttl_body, 0)

    return k(kid, cat, dif, tags_flat, ttok_t, ktab, ctab, dtab, tgtab, etab)


_LOG_MAX = float(np.log1p(100.0))


def _tc_body(ttl_ref, k_ref, c_ref, d_ref, tgr_ref, tags_ref, lens_ref,
             lc_ref, rt_ref, du_ref, dp_ref,
             ltab_ref, rtab_ref, dtab_ref, ftab_ref,
             wih_ref, bih_ref, whf_ref, whb_ref,
             w1_ref, b1_ref, g1_ref, be1_ref,
             w2_ref, b2_ref, g2_ref, be2_ref,
             out_ref, x_ref, hf_ref, cf_ref, hb_ref, cb_ref):
    emb = ttl_ref[...].reshape(TL * BT, TD)
    x_ref[...] = (jnp.dot(emb, wih_ref[...], preferred_element_type=_f32)
                  + bih_ref[...])

    lens = jnp.maximum(lens_ref[...], 1)

    zero = jnp.zeros((BT, HID), _f32)
    hf_ref[...] = zero
    cf_ref[...] = zero
    hb_ref[...] = zero
    cb_ref[...] = zero
    whf = whf_ref[...]
    whb = whb_ref[...]

    def sig(v):
        return 0.5 * jnp.tanh(0.5 * v) + 0.5

    def step(s, carry):
        tb = TL - 1 - s
        gf = (x_ref[pl.ds(s * BT, BT), 0:4 * HID]
              + jnp.dot(hf_ref[...], whf, preferred_element_type=_f32))
        i1, f1, g1_, o1 = jnp.split(gf, 4, axis=-1)
        cfn = sig(f1) * cf_ref[...] + sig(i1) * jnp.tanh(g1_)
        hfn = sig(o1) * jnp.tanh(cfn)
        mf = (s < lens).astype(_f32)
        hf_ref[...] = mf * hfn + (1.0 - mf) * hf_ref[...]
        cf_ref[...] = mf * cfn + (1.0 - mf) * cf_ref[...]
        gb = (x_ref[pl.ds(tb * BT, BT), 4 * HID:8 * HID]
              + jnp.dot(hb_ref[...], whb, preferred_element_type=_f32))
        i2, f2, g2_, o2 = jnp.split(gb, 4, axis=-1)
        cbn = sig(f2) * cb_ref[...] + sig(i2) * jnp.tanh(g2_)
        hbn = sig(o2) * jnp.tanh(cbn)
        mb = (tb < lens).astype(_f32)
        hb_ref[...] = mb * hbn + (1.0 - mb) * hb_ref[...]
        cb_ref[...] = mb * cbn + (1.0 - mb) * cb_ref[...]
        return carry

    lax.fori_loop(0, TL, step, 0)

    tsum = tgr_ref[:, 0, :]
    for t in range(1, TAGS):
        tsum = tsum + tgr_ref[:, t, :]
    tcnt = jnp.sum((tags_ref[...] != 0).astype(_f32), axis=1, keepdims=True)
    tvec = tsum / (tcnt + 1e-8)

    def bucket_emb(v, nb, use_log, tab_ref):
        if use_log:
            x = jnp.log(1.0 + jnp.maximum(v, 0.0))
            maxv = _LOG_MAX
        else:
            x = v
            maxv = 100.0
        idx = (x / (maxv + 1e-8) * nb).astype(jnp.int32)
        idx = jnp.clip(idx, 0, nb - 1)
        rows = tab_ref.shape[0]
        oh = (idx == lax.broadcasted_iota(jnp.int32, (BT, rows), 1)).astype(_f32)
        return jnp.dot(oh, tab_ref[...], preferred_element_type=_f32)

    le = bucket_emb(lc_ref[...], 20, True, ltab_ref)
    re_ = bucket_emb(rt_ref[...], 10, False, rtab_ref)
    de = bucket_emb(du_ref[...], 15, True, dtab_ref)
    fe = bucket_emb(dp_ref[...], 30, True, ftab_ref)

    feat = jnp.concatenate(
        [k_ref[...], c_ref[...], d_ref[...], tvec,
         hf_ref[...], hb_ref[...], le, re_, de, fe], axis=1)

    def ln(x, g, b):
        mu = jnp.mean(x, axis=-1, keepdims=True)
        var = jnp.mean((x - mu) ** 2, axis=-1, keepdims=True)
        return (x - mu) / jnp.sqrt(var + 1e-5) * g + b

    h1 = jnp.dot(feat, w1_ref[...], preferred_element_type=_f32) + b1_ref[...]
    h1 = jnp.maximum(ln(h1, g1_ref[...], be1_ref[...]), 0.0)
    o = jnp.dot(h1, w2_ref[...], preferred_element_type=_f32) + b2_ref[...]
    out_ref[...] = ln(o, g2_ref[...], be2_ref[...])


def _dense(ttl3, k_rows, c_rows, d_rows, tgr, tags_i, lens2,
           lc2, rt2, du2, dp2, ltab, rtab, dtab, ftab,
           wih, bih, whf, whb, w1, b1, g1, be1, w2, b2, g2, be2):
    def tile(shape, imap):
        return pl.BlockSpec(shape, imap)

    full2 = lambda a: pl.BlockSpec(a.shape, lambda i: (0, 0))
    in_specs = [
        tile((TL, BT, TD), lambda i: (0, i, 0)),
        tile((BT, ED), lambda i: (i, 0)),
        tile((BT, ED), lambda i: (i, 0)),
        tile((BT, ED), lambda i: (i, 0)),
        tile((BT, TAGS, ED), lambda i: (i, 0, 0)),
        tile((BT, TAGS), lambda i: (i, 0)),
        tile((BT, 1), lambda i: (i, 0)),
        tile((BT, 1), lambda i: (i, 0)),
        tile((BT, 1), lambda i: (i, 0)),
        tile((BT, 1), lambda i: (i, 0)),
        tile((BT, 1), lambda i: (i, 0)),
        full2(ltab), full2(rtab), full2(dtab), full2(ftab),
        full2(wih), full2(bih), full2(whf), full2(whb),
        full2(w1), full2(b1), full2(g1), full2(be1),
        full2(w2), full2(b2), full2(g2), full2(be2),
    ]
    return pl.pallas_call(
        _tc_body,
        grid=(NB,),
        in_specs=in_specs,
        out_specs=pl.BlockSpec((BT, ED), lambda i: (i, 0)),
        out_shape=jax.ShapeDtypeStruct((B, ED), _f32),
        scratch_shapes=[
            pltpu.VMEM((TL * BT, 8 * HID), _f32),
            pltpu.VMEM((BT, HID), _f32),
            pltpu.VMEM((BT, HID), _f32),
            pltpu.VMEM((BT, HID), _f32),
            pltpu.VMEM((BT, HID), _f32),
        ],
    )(ttl3, k_rows, c_rows, d_rows, tgr, tags_i, lens2,
      lc2, rt2, du2, dp2, ltab, rtab, dtab, ftab,
      wih, bih, whf, whb, w1, b1, g1, be1, w2, b2, g2, be2)


def kernel(knowledge_id, category, difficulty, tags, title_tokens,
           title_lengths, learner_count, rating, duration,
           days_since_publish, knowledge_table, category_table,
           difficulty_table, tag_table, learner_table, rating_table,
           duration_table, freshness_table, title_emb,
           W_ih_f, W_hh_f, b_ih_f, b_hh_f, W_ih_b, W_hh_b, b_ih_b, b_hh_b,
           W1, b1, g1, be1, W2, b2, g2, be2):
    kid = knowledge_id.astype(jnp.int32)
    cat = category.astype(jnp.int32)
    dif = difficulty.astype(jnp.int32)
    tags_i = tags.astype(jnp.int32)
    ttok = title_tokens.astype(jnp.int32)

    tags_flat = tags_i.reshape(B * TAGS)
    ttok_t = ttok.T.reshape(TL * B)

    k_rows, c_rows, d_rows, tag_rows, ttl_rows = _sc_gather(
        kid, cat, dif, tags_flat, ttok_t,
        knowledge_table, category_table, difficulty_table, tag_table,
        title_emb)

    ttl3 = ttl_rows.reshape(TL, B, TD)
    tgr = tag_rows.reshape(B, TAGS, ED)

    wih = jnp.concatenate([W_ih_f.T, W_ih_b.T], axis=1)
    bih = jnp.concatenate([b_ih_f + b_hh_f, b_ih_b + b_hh_b]).reshape(1, 8 * HID)
    whf = W_hh_f.T
    whb = W_hh_b.T

    lens2 = jnp.clip(title_lengths.astype(jnp.int32), 1, TL).reshape(B, 1)
    lc2 = learner_count.astype(_f32).reshape(B, 1)
    rt2 = rating.astype(_f32).reshape(B, 1)
    du2 = duration.astype(_f32).reshape(B, 1)
    dp2 = days_since_publish.astype(_f32).reshape(B, 1)

    return _dense(
        ttl3, k_rows, c_rows, d_rows, tgr, tags_i, lens2,
        lc2, rt2, du2, dp2,
        learner_table, rating_table, duration_table, freshness_table,
        wih, bih, whf, whb,
        W1, b1.reshape(1, -1), g1.reshape(1, -1), be1.reshape(1, -1),
        W2, b2.reshape(1, -1), g2.reshape(1, -1), be2.reshape(1, -1))


# speedup vs baseline: 2.5400x; 1.2730x over previous
import functools

import numpy as np
import jax
import jax.numpy as jnp
from jax import lax
from jax.experimental import pallas as pl
from jax.experimental.pallas import tpu as pltpu
from jax.experimental.pallas import tpu_sc as plsc

B = 4096
ED = 128
TAGS = 10
TL = 50
HID = 64
TD = 64
FIN = ED * 8 + 2 * HID

NC = 2
NS = 16
NW = NC * NS
PW = B // NW
CH = 128

_f32 = jnp.float32

BT = 256
NB = B // BT


def _sc_gather(kid, cat, dif, tags_flat, ttok_t, ktab, ctab, dtab, tgtab, etab):
    mesh = plsc.VectorSubcoreMesh(core_axis_name="c", subcore_axis_name="s")

    @functools.partial(
        pl.kernel,
        out_type=[
            jax.ShapeDtypeStruct((B, ED), _f32),
            jax.ShapeDtypeStruct((B, ED), _f32),
            jax.ShapeDtypeStruct((B, ED), _f32),
            jax.ShapeDtypeStruct((B * TAGS, ED), _f32),
            jax.ShapeDtypeStruct((TL * B, TD), _f32),
        ],
        mesh=mesh,
        compiler_params=pltpu.CompilerParams(use_tc_tiling_on_sc=False),
        scratch_types=[
            pltpu.VMEM((CH,), jnp.int32),
            pltpu.VMEM((CH, ED), _f32),
            pltpu.VMEM((CH, TD), _f32),
            pltpu.SemaphoreType.DMA,
        ],
    )
    def k(kid_h, cat_h, dif_h, tags_h, ttok_h, ktab_h, ctab_h, dtab_h,
          tgtab_h, etab_h, ko_h, co_h, do_h, tgo_h, tto_h,
          idx_v, rows_v, rows64_v, sem):
        cid = lax.axis_index("c")
        sid = lax.axis_index("s")
        wid = sid * NC + cid
        base = wid * PW

        def gather128(src_idx_h, table_h, out_h, off):
            pltpu.sync_copy(src_idx_h.at[pl.ds(off, CH)], idx_v)
            pltpu.async_copy(table_h.at[idx_v], rows_v, sem).wait()
            pltpu.sync_copy(rows_v, out_h.at[pl.ds(off, CH)])

        gather128(kid_h, ktab_h, ko_h, base)
        gather128(cat_h, ctab_h, co_h, base)
        gather128(dif_h, dtab_h, do_h, base)

        def tag_body(j, carry):
            gather128(tags_h, tgtab_h, tgo_h, base * TAGS + j * CH)
            return carry
        lax.fori_loop(0, PW * TAGS // CH, tag_body, 0)

        tbase = wid * (TL * B // NW)

        def ttl_body(j, carry):
            off = tbase + j * CH
            pltpu.sync_copy(ttok_h.at[pl.ds(off, CH)], idx_v)
            pltpu.async_copy(etab_h.at[idx_v], rows64_v, sem).wait()
            pltpu.sync_copy(rows64_v, tto_h.at[pl.ds(off, CH)])
            return carry
        lax.fori_loop(0, TL * B // NW // CH, ttl_body, 0)

    return k(kid, cat, dif, tags_flat, ttok_t, ktab, ctab, dtab, tgtab, etab)


_LOG_MAX = float(np.log1p(100.0))


def _tc_body(ttl_ref, k_ref, c_ref, d_ref, tgr_ref, tags_ref, lens_ref,
             lc_ref, rt_ref, du_ref, dp_ref,
             ltab_ref, rtab_ref, dtab_ref, ftab_ref,
             wih_ref, bih_ref, whf_ref, whb_ref,
             w1_ref, b1_ref, g1_ref, be1_ref,
             w2_ref, b2_ref, g2_ref, be2_ref,
             out_ref, x_ref, hf_ref, cf_ref, hb_ref, cb_ref):
    lens = jnp.maximum(lens_ref[...], 1)
    maxlen = jnp.max(lens)

    emb = ttl_ref[...].reshape(TL * BT, TD)
    wih = wih_ref[...]
    bih = bih_ref[...]
    XCH = 13
    for c0 in range(0, TL, XCH):
        c1 = min(c0 + XCH, TL)

        @pl.when(c0 < maxlen)
        def _():
            x_ref[pl.ds(c0 * BT, (c1 - c0) * BT), :] = (
                jnp.dot(emb[c0 * BT:c1 * BT, :], wih,
                        preferred_element_type=_f32) + bih)

    zero = jnp.zeros((BT, HID), _f32)
    hf_ref[...] = zero
    cf_ref[...] = zero
    hb_ref[...] = zero
    cb_ref[...] = zero
    whf = whf_ref[...]
    whb = whb_ref[...]

    def sig(v):
        return 0.5 * jnp.tanh(0.5 * v) + 0.5

    def step(s, carry):
        tb = maxlen - 1 - s
        gf = (x_ref[pl.ds(s * BT, BT), 0:4 * HID]
              + jnp.dot(hf_ref[...], whf, preferred_element_type=_f32))
        i1, f1, g1_, o1 = jnp.split(gf, 4, axis=-1)
        cfn = sig(f1) * cf_ref[...] + sig(i1) * jnp.tanh(g1_)
        hfn = sig(o1) * jnp.tanh(cfn)
        mf = (s < lens).astype(_f32)
        hf_ref[...] = mf * hfn + (1.0 - mf) * hf_ref[...]
        cf_ref[...] = mf * cfn + (1.0 - mf) * cf_ref[...]
        gb = (x_ref[pl.ds(tb * BT, BT), 4 * HID:8 * HID]
              + jnp.dot(hb_ref[...], whb, preferred_element_type=_f32))
        i2, f2, g2_, o2 = jnp.split(gb, 4, axis=-1)
        cbn = sig(f2) * cb_ref[...] + sig(i2) * jnp.tanh(g2_)
        hbn = sig(o2) * jnp.tanh(cbn)
        mb = (tb < lens).astype(_f32)
        hb_ref[...] = mb * hbn + (1.0 - mb) * hb_ref[...]
        cb_ref[...] = mb * cbn + (1.0 - mb) * cb_ref[...]
        return carry

    lax.fori_loop(0, maxlen, step, 0)

    tsum = tgr_ref[:, 0, :]
    for t in range(1, TAGS):
        tsum = tsum + tgr_ref[:, t, :]
    tcnt = jnp.sum((tags_ref[...] != 0).astype(_f32), axis=1, keepdims=True)
    tvec = tsum / (tcnt + 1e-8)

    def bucket_emb(v, nb, use_log, tab_ref):
        if use_log:
            x = jnp.log(1.0 + jnp.maximum(v, 0.0))
            maxv = _LOG_MAX
        else:
            x = v
            maxv = 100.0
        idx = (x / (maxv + 1e-8) * nb).astype(jnp.int32)
        idx = jnp.clip(idx, 0, nb - 1)
        rows = tab_ref.shape[0]
        oh = (idx == lax.broadcasted_iota(jnp.int32, (BT, rows), 1)).astype(_f32)
        return jnp.dot(oh, tab_ref[...], preferred_element_type=_f32)

    le = bucket_emb(lc_ref[...], 20, True, ltab_ref)
    re_ = bucket_emb(rt_ref[...], 10, False, rtab_ref)
    de = bucket_emb(du_ref[...], 15, True, dtab_ref)
    fe = bucket_emb(dp_ref[...], 30, True, ftab_ref)

    feat = jnp.concatenate(
        [k_ref[...], c_ref[...], d_ref[...], tvec,
         hf_ref[...], hb_ref[...], le, re_, de, fe], axis=1)

    def ln(x, g, b):
        mu = jnp.mean(x, axis=-1, keepdims=True)
        var = jnp.mean((x - mu) ** 2, axis=-1, keepdims=True)
        return (x - mu) / jnp.sqrt(var + 1e-5) * g + b

    h1 = jnp.dot(feat, w1_ref[...], preferred_element_type=_f32) + b1_ref[...]
    h1 = jnp.maximum(ln(h1, g1_ref[...], be1_ref[...]), 0.0)
    o = jnp.dot(h1, w2_ref[...], preferred_element_type=_f32) + b2_ref[...]
    out_ref[...] = ln(o, g2_ref[...], be2_ref[...])


def _dense(ttl3, k_rows, c_rows, d_rows, tgr, tags_i, lens2,
           lc2, rt2, du2, dp2, ltab, rtab, dtab, ftab,
           wih, bih, whf, whb, w1, b1, g1, be1, w2, b2, g2, be2):
    def tile(shape, imap):
        return pl.BlockSpec(shape, imap)

    full2 = lambda a: pl.BlockSpec(a.shape, lambda i: (0, 0))
    in_specs = [
        tile((TL, BT, TD), lambda i: (0, i, 0)),
        tile((BT, ED), lambda i: (i, 0)),
        tile((BT, ED), lambda i: (i, 0)),
        tile((BT, ED), lambda i: (i, 0)),
        tile((BT, TAGS, ED), lambda i: (i, 0, 0)),
        tile((BT, TAGS), lambda i: (i, 0)),
        tile((BT, 1), lambda i: (i, 0)),
        tile((BT, 1), lambda i: (i, 0)),
        tile((BT, 1), lambda i: (i, 0)),
        tile((BT, 1), lambda i: (i, 0)),
        tile((BT, 1), lambda i: (i, 0)),
        full2(ltab), full2(rtab), full2(dtab), full2(ftab),
        full2(wih), full2(bih), full2(whf), full2(whb),
        full2(w1), full2(b1), full2(g1), full2(be1),
        full2(w2), full2(b2), full2(g2), full2(be2),
    ]
    return pl.pallas_call(
        _tc_body,
        grid=(NB,),
        in_specs=in_specs,
        out_specs=pl.BlockSpec((BT, ED), lambda i: (i, 0)),
        out_shape=jax.ShapeDtypeStruct((B, ED), _f32),
        scratch_shapes=[
            pltpu.VMEM((TL * BT, 8 * HID), _f32),
            pltpu.VMEM((BT, HID), _f32),
            pltpu.VMEM((BT, HID), _f32),
            pltpu.VMEM((BT, HID), _f32),
            pltpu.VMEM((BT, HID), _f32),
        ],
    )(ttl3, k_rows, c_rows, d_rows, tgr, tags_i, lens2,
      lc2, rt2, du2, dp2, ltab, rtab, dtab, ftab,
      wih, bih, whf, whb, w1, b1, g1, be1, w2, b2, g2, be2)


def kernel(knowledge_id, category, difficulty, tags, title_tokens,
           title_lengths, learner_count, rating, duration,
           days_since_publish, knowledge_table, category_table,
           difficulty_table, tag_table, learner_table, rating_table,
           duration_table, freshness_table, title_emb,
           W_ih_f, W_hh_f, b_ih_f, b_hh_f, W_ih_b, W_hh_b, b_ih_b, b_hh_b,
           W1, b1, g1, be1, W2, b2, g2, be2):
    lens_i = jnp.clip(title_lengths.astype(jnp.int32), 1, TL)
    perm = jnp.argsort(lens_i)

    kid = knowledge_id.astype(jnp.int32)[perm]
    cat = category.astype(jnp.int32)[perm]
    dif = difficulty.astype(jnp.int32)[perm]
    tags_i = tags.astype(jnp.int32)[perm]
    ttok = title_tokens.astype(jnp.int32)[perm]

    tags_flat = tags_i.reshape(B * TAGS)
    ttok_t = ttok.T.reshape(TL * B)

    k_rows, c_rows, d_rows, tag_rows, ttl_rows = _sc_gather(
        kid, cat, dif, tags_flat, ttok_t,
        knowledge_table, category_table, difficulty_table, tag_table,
        title_emb)

    ttl3 = ttl_rows.reshape(TL, B, TD)
    tgr = tag_rows.reshape(B, TAGS, ED)

    wih = jnp.concatenate([W_ih_f.T, W_ih_b.T], axis=1)
    bih = jnp.concatenate([b_ih_f + b_hh_f, b_ih_b + b_hh_b]).reshape(1, 8 * HID)
    whf = W_hh_f.T
    whb = W_hh_b.T

    lens2 = lens_i[perm].reshape(B, 1)
    lc2 = learner_count.astype(_f32)[perm].reshape(B, 1)
    rt2 = rating.astype(_f32)[perm].reshape(B, 1)
    du2 = duration.astype(_f32)[perm].reshape(B, 1)
    dp2 = days_since_publish.astype(_f32)[perm].reshape(B, 1)

    out_s = _dense(
        ttl3, k_rows, c_rows, d_rows, tgr, tags_i, lens2,
        lc2, rt2, du2, dp2,
        learner_table, rating_table, duration_table, freshness_table,
        wih, bih, whf, whb,
        W1, b1.reshape(1, -1), g1.reshape(1, -1), be1.reshape(1, -1),
        W2, b2.reshape(1, -1), g2.reshape(1, -1), be2.reshape(1, -1))
    return jnp.zeros((B, ED), _f32).at[perm].set(out_s)


# speedup vs baseline: 3.2357x; 1.2739x over previous
import functools

import numpy as np
import jax
import jax.numpy as jnp
from jax import lax
from jax.experimental import pallas as pl
from jax.experimental.pallas import tpu as pltpu
from jax.experimental.pallas import tpu_sc as plsc

B = 4096
ED = 128
TAGS = 10
TL = 50
HID = 64
TD = 64
FIN = ED * 8 + 2 * HID

NC = 2
NS = 16
NW = NC * NS
PW = B // NW
CH = 128

_f32 = jnp.float32

BT = 256
NB = B // BT


def _sc_gather(kid, cat, dif, tags_flat, ttok_t, ktab, ctab, dtab, tgtab, etab):
    mesh = plsc.VectorSubcoreMesh(core_axis_name="c", subcore_axis_name="s")

    @functools.partial(
        pl.kernel,
        out_type=[
            jax.ShapeDtypeStruct((B, ED), _f32),
            jax.ShapeDtypeStruct((B, ED), _f32),
            jax.ShapeDtypeStruct((B, ED), _f32),
            jax.ShapeDtypeStruct((B * TAGS, ED), _f32),
            jax.ShapeDtypeStruct((TL * B, TD), _f32),
        ],
        mesh=mesh,
        compiler_params=pltpu.CompilerParams(use_tc_tiling_on_sc=False),
        scratch_types=[
            pltpu.VMEM((CH,), jnp.int32),
            pltpu.VMEM((CH, ED), _f32),
            pltpu.VMEM((CH, TD), _f32),
            pltpu.SemaphoreType.DMA,
        ],
    )
    def k(kid_h, cat_h, dif_h, tags_h, ttok_h, ktab_h, ctab_h, dtab_h,
          tgtab_h, etab_h, ko_h, co_h, do_h, tgo_h, tto_h,
          idx_v, rows_v, rows64_v, sem):
        cid = lax.axis_index("c")
        sid = lax.axis_index("s")
        wid = sid * NC + cid
        base = wid * PW

        def gather128(src_idx_h, table_h, out_h, off):
            pltpu.sync_copy(src_idx_h.at[pl.ds(off, CH)], idx_v)
            pltpu.async_copy(table_h.at[idx_v], rows_v, sem).wait()
            pltpu.sync_copy(rows_v, out_h.at[pl.ds(off, CH)])

        gather128(kid_h, ktab_h, ko_h, base)
        gather128(cat_h, ctab_h, co_h, base)
        gather128(dif_h, dtab_h, do_h, base)

        def tag_body(j, carry):
            gather128(tags_h, tgtab_h, tgo_h, base * TAGS + j * CH)
            return carry
        lax.fori_loop(0, PW * TAGS // CH, tag_body, 0)

        tbase = wid * (TL * B // NW)

        def ttl_body(j, carry):
            off = tbase + j * CH
            pltpu.sync_copy(ttok_h.at[pl.ds(off, CH)], idx_v)
            pltpu.async_copy(etab_h.at[idx_v], rows64_v, sem).wait()
            pltpu.sync_copy(rows64_v, tto_h.at[pl.ds(off, CH)])
            return carry
        lax.fori_loop(0, TL * B // NW // CH, ttl_body, 0)

    return k(kid, cat, dif, tags_flat, ttok_t, ktab, ctab, dtab, tgtab, etab)


_LOG_MAX = float(np.log1p(100.0))


def _tc_body(ttl_ref, k_ref, c_ref, d_ref, tgr_ref, tags_ref, lens_ref,
             lc_ref, rt_ref, du_ref, dp_ref,
             ltab_ref, rtab_ref, dtab_ref, ftab_ref,
             wih_ref, bih_ref, whh_ref,
             w1_ref, b1_ref, g1_ref, be1_ref,
             w2_ref, b2_ref, g2_ref, be2_ref,
             out_ref, x_ref, h_ref, c_ref2):
    lens = jnp.maximum(lens_ref[...], 1)
    maxlen = jnp.max(lens)

    emb = ttl_ref[...].reshape(TL * BT, TD).astype(jnp.bfloat16)
    wih = wih_ref[...].astype(jnp.bfloat16)
    bih = bih_ref[...]
    XCH = 13
    for c0 in range(0, TL, XCH):
        c1 = min(c0 + XCH, TL)

        @pl.when(c0 < maxlen)
        def _():
            x_ref[pl.ds(c0 * BT, (c1 - c0) * BT), :] = (
                jnp.dot(emb[c0 * BT:c1 * BT, :], wih,
                        preferred_element_type=_f32) + bih)

    h_ref[...] = jnp.zeros((BT, 2 * HID), _f32)
    c_ref2[...] = jnp.zeros((BT, 2 * HID), _f32)
    whh = whh_ref[...].astype(jnp.bfloat16)
    dir512 = ((lax.broadcasted_iota(jnp.int32, (1, 8 * HID), 1) % (2 * HID))
              < HID).astype(_f32)
    dir128 = (lax.broadcasted_iota(jnp.int32, (1, 2 * HID), 1)
              < HID).astype(_f32)

    def sig(v):
        return 0.5 * jnp.tanh(0.5 * v) + 0.5

    def step(s, carry):
        tb = maxlen - 1 - s
        xf = x_ref[pl.ds(s * BT, BT), :]
        xb = x_ref[pl.ds(tb * BT, BT), :]
        x_t = xb + dir512 * (xf - xb)
        gates = x_t + jnp.dot(h_ref[...].astype(jnp.bfloat16), whh,
                              preferred_element_type=_f32)
        gi = sig(gates[:, 0:2 * HID])
        gf = sig(gates[:, 2 * HID:4 * HID])
        gg = jnp.tanh(gates[:, 4 * HID:6 * HID])
        go = sig(gates[:, 6 * HID:8 * HID])
        c_new = gf * c_ref2[...] + gi * gg
        h_new = go * jnp.tanh(c_new)
        mf = (s < lens).astype(_f32)
        mb2 = (tb < lens).astype(_f32)
        m = mb2 + dir128 * (mf - mb2)
        h_ref[...] = h_ref[...] + m * (h_new - h_ref[...])
        c_ref2[...] = c_ref2[...] + m * (c_new - c_ref2[...])
        return carry

    lax.fori_loop(0, maxlen, step, 0)

    tsum = tgr_ref[:, 0, :]
    for t in range(1, TAGS):
        tsum = tsum + tgr_ref[:, t, :]
    tcnt = jnp.sum((tags_ref[...] != 0).astype(_f32), axis=1, keepdims=True)
    tvec = tsum / (tcnt + 1e-8)

    def bucket_emb(v, nb, use_log, tab_ref):
        if use_log:
            x = jnp.log(1.0 + jnp.maximum(v, 0.0))
            maxv = _LOG_MAX
        else:
            x = v
            maxv = 100.0
        idx = (x / (maxv + 1e-8) * nb).astype(jnp.int32)
        idx = jnp.clip(idx, 0, nb - 1)
        rows = tab_ref.shape[0]
        oh = (idx == lax.broadcasted_iota(jnp.int32, (BT, rows), 1)).astype(_f32)
        return jnp.dot(oh, tab_ref[...], preferred_element_type=_f32)

    le = bucket_emb(lc_ref[...], 20, True, ltab_ref)
    re_ = bucket_emb(rt_ref[...], 10, False, rtab_ref)
    de = bucket_emb(du_ref[...], 15, True, dtab_ref)
    fe = bucket_emb(dp_ref[...], 30, True, ftab_ref)

    feat = jnp.concatenate(
        [k_ref[...], c_ref[...], d_ref[...], tvec,
         h_ref[...], le, re_, de, fe], axis=1)

    def ln(x, g, b):
        mu = jnp.mean(x, axis=-1, keepdims=True)
        var = jnp.mean((x - mu) ** 2, axis=-1, keepdims=True)
        return (x - mu) / jnp.sqrt(var + 1e-5) * g + b

    h1 = jnp.dot(feat.astype(jnp.bfloat16), w1_ref[...].astype(jnp.bfloat16),
                 preferred_element_type=_f32) + b1_ref[...]
    h1 = jnp.maximum(ln(h1, g1_ref[...], be1_ref[...]), 0.0)
    o = jnp.dot(h1.astype(jnp.bfloat16), w2_ref[...].astype(jnp.bfloat16),
                preferred_element_type=_f32) + b2_ref[...]
    out_ref[...] = ln(o, g2_ref[...], be2_ref[...])


def _dense(ttl3, k_rows, c_rows, d_rows, tgr, tags_i, lens2,
           lc2, rt2, du2, dp2, ltab, rtab, dtab, ftab,
           wih, bih, whh, w1, b1, g1, be1, w2, b2, g2, be2):
    def tile(shape, imap):
        return pl.BlockSpec(shape, imap)

    full2 = lambda a: pl.BlockSpec(a.shape, lambda i: (0, 0))
    in_specs = [
        tile((TL, BT, TD), lambda i: (0, i, 0)),
        tile((BT, ED), lambda i: (i, 0)),
        tile((BT, ED), lambda i: (i, 0)),
        tile((BT, ED), lambda i: (i, 0)),
        tile((BT, TAGS, ED), lambda i: (i, 0, 0)),
        tile((BT, TAGS), lambda i: (i, 0)),
        tile((BT, 1), lambda i: (i, 0)),
        tile((BT, 1), lambda i: (i, 0)),
        tile((BT, 1), lambda i: (i, 0)),
        tile((BT, 1), lambda i: (i, 0)),
        tile((BT, 1), lambda i: (i, 0)),
        full2(ltab), full2(rtab), full2(dtab), full2(ftab),
        full2(wih), full2(bih), full2(whh),
        full2(w1), full2(b1), full2(g1), full2(be1),
        full2(w2), full2(b2), full2(g2), full2(be2),
    ]
    return pl.pallas_call(
        _tc_body,
        grid=(NB,),
        in_specs=in_specs,
        out_specs=pl.BlockSpec((BT, ED), lambda i: (i, 0)),
        out_shape=jax.ShapeDtypeStruct((B, ED), _f32),
        scratch_shapes=[
            pltpu.VMEM((TL * BT, 8 * HID), _f32),
            pltpu.VMEM((BT, 2 * HID), _f32),
            pltpu.VMEM((BT, 2 * HID), _f32),
        ],
    )(ttl3, k_rows, c_rows, d_rows, tgr, tags_i, lens2,
      lc2, rt2, du2, dp2, ltab, rtab, dtab, ftab,
      wih, bih, whh, w1, b1, g1, be1, w2, b2, g2, be2)


def kernel(knowledge_id, category, difficulty, tags, title_tokens,
           title_lengths, learner_count, rating, duration,
           days_since_publish, knowledge_table, category_table,
           difficulty_table, tag_table, learner_table, rating_table,
           duration_table, freshness_table, title_emb,
           W_ih_f, W_hh_f, b_ih_f, b_hh_f, W_ih_b, W_hh_b, b_ih_b, b_hh_b,
           W1, b1, g1, be1, W2, b2, g2, be2):
    lens_i = jnp.clip(title_lengths.astype(jnp.int32), 1, TL)
    perm = jnp.argsort(lens_i)

    kid = knowledge_id.astype(jnp.int32)[perm]
    cat = category.astype(jnp.int32)[perm]
    dif = difficulty.astype(jnp.int32)[perm]
    tags_i = tags.astype(jnp.int32)[perm]
    ttok = title_tokens.astype(jnp.int32)[perm]

    tags_flat = tags_i.reshape(B * TAGS)
    ttok_t = ttok.T.reshape(TL * B)

    k_rows, c_rows, d_rows, tag_rows, ttl_rows = _sc_gather(
        kid, cat, dif, tags_flat, ttok_t,
        knowledge_table, category_table, difficulty_table, tag_table,
        title_emb)

    ttl3 = ttl_rows.reshape(TL, B, TD)
    tgr = tag_rows.reshape(B, TAGS, ED)

    def gate_major(wf_t, wb_t):
        kdim = wf_t.shape[0]
        return jnp.stack(
            [wf_t.reshape(kdim, 4, HID), wb_t.reshape(kdim, 4, HID)],
            axis=2).reshape(kdim, 8 * HID)

    wih = gate_major(W_ih_f.T, W_ih_b.T)
    bih = jnp.stack(
        [(b_ih_f + b_hh_f).reshape(4, HID), (b_ih_b + b_hh_b).reshape(4, HID)],
        axis=1).reshape(1, 8 * HID)
    z64 = jnp.zeros((HID, 4, HID), _f32)
    top = jnp.stack([W_hh_f.T.reshape(HID, 4, HID), z64], axis=2)
    bot = jnp.stack([z64, W_hh_b.T.reshape(HID, 4, HID)], axis=2)
    whh = jnp.concatenate(
        [top.reshape(HID, 8 * HID), bot.reshape(HID, 8 * HID)], axis=0)

    lens2 = lens_i[perm].reshape(B, 1)
    lc2 = learner_count.astype(_f32)[perm].reshape(B, 1)
    rt2 = rating.astype(_f32)[perm].reshape(B, 1)
    du2 = duration.astype(_f32)[perm].reshape(B, 1)
    dp2 = days_since_publish.astype(_f32)[perm].reshape(B, 1)

    out_s = _dense(
        ttl3, k_rows, c_rows, d_rows, tgr, tags_i, lens2,
        lc2, rt2, du2, dp2,
        learner_table, rating_table, duration_table, freshness_table,
        wih, bih, whh,
        W1, b1.reshape(1, -1), g1.reshape(1, -1), be1.reshape(1, -1),
        W2, b2.reshape(1, -1), g2.reshape(1, -1), be2.reshape(1, -1))
    return jnp.zeros((B, ED), _f32).at[perm].set(out_s)


# speedup vs baseline: 3.4859x; 1.0773x over previous
import functools

import numpy as np
import jax
import jax.numpy as jnp
from jax import lax
from jax.experimental import pallas as pl
from jax.experimental.pallas import tpu as pltpu
from jax.experimental.pallas import tpu_sc as plsc

B = 4096
ED = 128
TAGS = 10
TL = 50
HID = 64
TD = 64
FIN = ED * 8 + 2 * HID

NC = 2
NS = 16
NW = NC * NS
PW = B // NW
CH = 128

_f32 = jnp.float32

BT = 256
NB = B // BT


TCH = TL * B // NW // CH
G = 4


def _sc_gather(kid, cat, dif, tags_flat, ttok_t, meta, ktab, ctab, dtab,
               tgtab, etab):
    mesh = plsc.VectorSubcoreMesh(core_axis_name="c", subcore_axis_name="s")

    @functools.partial(
        pl.kernel,
        out_type=[
            jax.ShapeDtypeStruct((B, ED), _f32),
            jax.ShapeDtypeStruct((B, ED), _f32),
            jax.ShapeDtypeStruct((B, ED), _f32),
            jax.ShapeDtypeStruct((B * TAGS, ED), _f32),
            jax.ShapeDtypeStruct((TL * B, TD), _f32),
        ],
        mesh=mesh,
        compiler_params=pltpu.CompilerParams(use_tc_tiling_on_sc=False),
        scratch_types=[
            pltpu.VMEM((16, 16), jnp.int32),
            pltpu.VMEM((TL * B // NW,), jnp.int32),
            pltpu.VMEM((PW * TAGS,), jnp.int32),
            pltpu.VMEM((CH,), jnp.int32),
            pltpu.VMEM((CH,), jnp.int32),
            pltpu.VMEM((CH,), jnp.int32),
            pltpu.VMEM((G * CH, ED), _f32),
            pltpu.VMEM((G * CH, TD), _f32),
            pltpu.SemaphoreType.DMA,
            pltpu.SemaphoreType.DMA,
        ],
    )
    def k(kid_h, cat_h, dif_h, tags_h, ttok_h, meta_h, ktab_h, ctab_h,
          dtab_h, tgtab_h, etab_h, ko_h, co_h, do_h, tgo_h, tto_h,
          meta_v, tidx_v, gidx_v, ki_v, ci_v, di_v, rows_v, rows64_v,
          semg, sems):
        cid = lax.axis_index("c")
        sid = lax.axis_index("s")
        wid = sid * NC + cid
        base = wid * PW
        tbase = wid * (TL * B // NW)

        hs = [
            pltpu.async_copy(meta_h.at[wid], meta_v, semg),
            pltpu.async_copy(ttok_h.at[pl.ds(tbase, TL * B // NW)], tidx_v,
                             semg),
            pltpu.async_copy(tags_h.at[pl.ds(base * TAGS, PW * TAGS)], gidx_v,
                             semg),
            pltpu.async_copy(kid_h.at[pl.ds(base, CH)], ki_v, semg),
            pltpu.async_copy(cat_h.at[pl.ds(base, CH)], ci_v, semg),
            pltpu.async_copy(dif_h.at[pl.ds(base, CH)], di_v, semg),
        ]
        for h in hs:
            h.wait()

        hs = [
            pltpu.async_copy(ktab_h.at[ki_v], rows_v.at[pl.ds(0, CH)], semg),
            pltpu.async_copy(ctab_h.at[ci_v], rows_v.at[pl.ds(CH, CH)], semg),
            pltpu.async_copy(dtab_h.at[di_v], rows_v.at[pl.ds(2 * CH, CH)],
                             semg),
        ]
        for h in hs:
            h.wait()
        hs = [
            pltpu.async_copy(rows_v.at[pl.ds(0, CH)],
                             ko_h.at[pl.ds(base, CH)], sems),
            pltpu.async_copy(rows_v.at[pl.ds(CH, CH)],
                             co_h.at[pl.ds(base, CH)], sems),
            pltpu.async_copy(rows_v.at[pl.ds(2 * CH, CH)],
                             do_h.at[pl.ds(base, CH)], sems),
        ]
        for h in hs:
            h.wait()

        def tag_group(g0, n):
            hs = [pltpu.async_copy(
                tgtab_h.at[gidx_v.at[pl.ds((g0 + u) * CH, CH)]],
                rows_v.at[pl.ds(u * CH, CH)], semg) for u in range(n)]
            for h in hs:
                h.wait()
            hs = [pltpu.async_copy(
                rows_v.at[pl.ds(u * CH, CH)],
                tgo_h.at[pl.ds(base * TAGS + (g0 + u) * CH, CH)], sems)
                for u in range(n)]
            for h in hs:
                h.wait()

        tag_group(0, G)
        tag_group(G, G)
        tag_group(2 * G, PW * TAGS // CH - 2 * G)

        ng = meta_v[0, :][0]

        def ttl_group(g, carry):
            ov = meta_v[g + 1, :]
            ghs = []
            for u in range(G):
                off = ov[u]
                loc = pl.multiple_of(off - tbase, CH)
                ghs.append(pltpu.async_copy(
                    etab_h.at[tidx_v.at[pl.ds(loc, CH)]],
                    rows64_v.at[pl.ds(u * CH, CH)], semg))
            for h in ghs:
                h.wait()
            shs = []
            for u in range(G):
                off = pl.multiple_of(ov[u], CH)
                shs.append(pltpu.async_copy(
                    rows64_v.at[pl.ds(u * CH, CH)],
                    tto_h.at[pl.ds(off, CH)], sems))
            for h in shs:
                h.wait()
            return carry

        lax.fori_loop(0, ng, ttl_group, 0)

    return k(kid, cat, dif, tags_flat, ttok_t, meta, ktab, ctab, dtab,
             tgtab, etab)


_LOG_MAX = float(np.log1p(100.0))


def _tc_body(ttl_ref, k_ref, c_ref, d_ref, tgr_ref, tags_ref, lens_ref,
             lc_ref, rt_ref, du_ref, dp_ref,
             ltab_ref, rtab_ref, dtab_ref, ftab_ref,
             wih_ref, bih_ref, whh_ref,
             w1_ref, b1_ref, g1_ref, be1_ref,
             w2_ref, b2_ref, g2_ref, be2_ref,
             out_ref, x_ref, h_ref, c_ref2):
    lens = jnp.maximum(lens_ref[...], 1)
    maxlen = jnp.max(lens)

    emb = ttl_ref[...].reshape(TL * BT, TD).astype(jnp.bfloat16)
    wih = wih_ref[...].astype(jnp.bfloat16)
    bih = bih_ref[...]
    XCH = 13
    for c0 in range(0, TL, XCH):
        c1 = min(c0 + XCH, TL)

        @pl.when(c0 < maxlen)
        def _():
            x_ref[pl.ds(c0 * BT, (c1 - c0) * BT), :] = (
                jnp.dot(emb[c0 * BT:c1 * BT, :], wih,
                        preferred_element_type=_f32) + bih)

    h_ref[...] = jnp.zeros((BT, 2 * HID), _f32)
    c_ref2[...] = jnp.zeros((BT, 2 * HID), _f32)
    whh = whh_ref[...].astype(jnp.bfloat16)
    dir512 = ((lax.broadcasted_iota(jnp.int32, (1, 8 * HID), 1) % (2 * HID))
              < HID).astype(_f32)
    dir128 = (lax.broadcasted_iota(jnp.int32, (1, 2 * HID), 1)
              < HID).astype(_f32)

    def sig(v):
        return 0.5 * jnp.tanh(0.5 * v) + 0.5

    def step(s, carry):
        tb = maxlen - 1 - s
        xf = x_ref[pl.ds(s * BT, BT), :]
        xb = x_ref[pl.ds(tb * BT, BT), :]
        x_t = xb + dir512 * (xf - xb)
        gates = x_t + jnp.dot(h_ref[...].astype(jnp.bfloat16), whh,
                              preferred_element_type=_f32)
        gi = sig(gates[:, 0:2 * HID])
        gf = sig(gates[:, 2 * HID:4 * HID])
        gg = jnp.tanh(gates[:, 4 * HID:6 * HID])
        go = sig(gates[:, 6 * HID:8 * HID])
        c_new = gf * c_ref2[...] + gi * gg
        h_new = go * jnp.tanh(c_new)
        mf = (s < lens).astype(_f32)
        mb2 = (tb < lens).astype(_f32)
        m = mb2 + dir128 * (mf - mb2)
        h_ref[...] = h_ref[...] + m * (h_new - h_ref[...])
        c_ref2[...] = c_ref2[...] + m * (c_new - c_ref2[...])
        return carry

    lax.fori_loop(0, maxlen, step, 0)

    tsum = tgr_ref[:, 0, :]
    for t in range(1, TAGS):
        tsum = tsum + tgr_ref[:, t, :]
    tcnt = jnp.sum((tags_ref[...] != 0).astype(_f32), axis=1, keepdims=True)
    tvec = tsum / (tcnt + 1e-8)

    def bucket_emb(v, nb, use_log, tab_ref):
        if use_log:
            x = jnp.log(1.0 + jnp.maximum(v, 0.0))
            maxv = _LOG_MAX
        else:
            x = v
            maxv = 100.0
        idx = (x / (maxv + 1e-8) * nb).astype(jnp.int32)
        idx = jnp.clip(idx, 0, nb - 1)
        rows = tab_ref.shape[0]
        oh = (idx == lax.broadcasted_iota(jnp.int32, (BT, rows), 1)).astype(_f32)
        return jnp.dot(oh, tab_ref[...], preferred_element_type=_f32)

    le = bucket_emb(lc_ref[...], 20, True, ltab_ref)
    re_ = bucket_emb(rt_ref[...], 10, False, rtab_ref)
    de = bucket_emb(du_ref[...], 15, True, dtab_ref)
    fe = bucket_emb(dp_ref[...], 30, True, ftab_ref)

    feat = jnp.concatenate(
        [k_ref[...], c_ref[...], d_ref[...], tvec,
         h_ref[...], le, re_, de, fe], axis=1)

    def ln(x, g, b):
        mu = jnp.mean(x, axis=-1, keepdims=True)
        var = jnp.mean((x - mu) ** 2, axis=-1, keepdims=True)
        return (x - mu) / jnp.sqrt(var + 1e-5) * g + b

    h1 = jnp.dot(feat.astype(jnp.bfloat16), w1_ref[...].astype(jnp.bfloat16),
                 preferred_element_type=_f32) + b1_ref[...]
    h1 = jnp.maximum(ln(h1, g1_ref[...], be1_ref[...]), 0.0)
    o = jnp.dot(h1.astype(jnp.bfloat16), w2_ref[...].astype(jnp.bfloat16),
                preferred_element_type=_f32) + b2_ref[...]
    out_ref[...] = ln(o, g2_ref[...], be2_ref[...])


def _dense(ttl3, k_rows, c_rows, d_rows, tgr, tags_i, lens2,
           lc2, rt2, du2, dp2, ltab, rtab, dtab, ftab,
           wih, bih, whh, w1, b1, g1, be1, w2, b2, g2, be2):
    def tile(shape, imap):
        return pl.BlockSpec(shape, imap)

    full2 = lambda a: pl.BlockSpec(a.shape, lambda i: (0, 0))
    in_specs = [
        tile((TL, BT, TD), lambda i: (0, i, 0)),
        tile((BT, ED), lambda i: (i, 0)),
        tile((BT, ED), lambda i: (i, 0)),
        tile((BT, ED), lambda i: (i, 0)),
        tile((BT, TAGS, ED), lambda i: (i, 0, 0)),
        tile((BT, TAGS), lambda i: (i, 0)),
        tile((BT, 1), lambda i: (i, 0)),
        tile((BT, 1), lambda i: (i, 0)),
        tile((BT, 1), lambda i: (i, 0)),
        tile((BT, 1), lambda i: (i, 0)),
        tile((BT, 1), lambda i: (i, 0)),
        full2(ltab), full2(rtab), full2(dtab), full2(ftab),
        full2(wih), full2(bih), full2(whh),
        full2(w1), full2(b1), full2(g1), full2(be1),
        full2(w2), full2(b2), full2(g2), full2(be2),
    ]
    return pl.pallas_call(
        _tc_body,
        grid=(NB,),
        in_specs=in_specs,
        out_specs=pl.BlockSpec((BT, ED), lambda i: (i, 0)),
        out_shape=jax.ShapeDtypeStruct((B, ED), _f32),
        scratch_shapes=[
            pltpu.VMEM((TL * BT, 8 * HID), _f32),
            pltpu.VMEM((BT, 2 * HID), _f32),
            pltpu.VMEM((BT, 2 * HID), _f32),
        ],
    )(ttl3, k_rows, c_rows, d_rows, tgr, tags_i, lens2,
      lc2, rt2, du2, dp2, ltab, rtab, dtab, ftab,
      wih, bih, whh, w1, b1, g1, be1, w2, b2, g2, be2)


def kernel(knowledge_id, category, difficulty, tags, title_tokens,
           title_lengths, learner_count, rating, duration,
           days_since_publish, knowledge_table, category_table,
           difficulty_table, tag_table, learner_table, rating_table,
           duration_table, freshness_table, title_emb,
           W_ih_f, W_hh_f, b_ih_f, b_hh_f, W_ih_b, W_hh_b, b_ih_b, b_hh_b,
           W1, b1, g1, be1, W2, b2, g2, be2):
    lens_i = jnp.clip(title_lengths.astype(jnp.int32), 1, TL)
    perm = jnp.argsort(lens_i)

    kid = knowledge_id.astype(jnp.int32)[perm]
    cat = category.astype(jnp.int32)[perm]
    dif = difficulty.astype(jnp.int32)[perm]
    tags_i = tags.astype(jnp.int32)[perm]
    ttok = title_tokens.astype(jnp.int32)[perm]

    tags_flat = tags_i.reshape(B * TAGS)
    ttok_t = ttok.T.reshape(TL * B)

    lens_s = lens_i[perm]
    c_all = jnp.arange(TL * B // CH, dtype=jnp.int32)
    t_of = (c_all * CH) // B
    bfirst = (c_all * CH) % B
    tile_last = (bfirst // BT) * BT + (BT - 1)
    needed = t_of < lens_s[tile_last]
    needed_w = needed.reshape(NW, TCH)
    order = jnp.argsort(jnp.where(needed_w, 0, 1), axis=1)
    offs = ((jnp.arange(NW, dtype=jnp.int32)[:, None] * TCH + order) * CH
            ).astype(jnp.int32)
    counts = needed_w.astype(jnp.int32).sum(axis=1)
    ngroups = (counts + (G - 1)) // G
    offs_pad = jnp.concatenate([offs, offs[:, -1:], offs[:, -1:]], axis=1)
    meta = (jnp.zeros((NW, 16, 16), jnp.int32)
            .at[:, 1:14, 0:G].set(offs_pad.reshape(NW, 13, G))
            .at[:, 0, 0].set(ngroups))

    k_rows, c_rows, d_rows, tag_rows, ttl_rows = _sc_gather(
        kid, cat, dif, tags_flat, ttok_t, meta,
        knowledge_table, category_table, difficulty_table, tag_table,
        title_emb)

    ttl3 = ttl_rows.reshape(TL, B, TD)
    tgr = tag_rows.reshape(B, TAGS, ED)

    def gate_major(wf_t, wb_t):
        kdim = wf_t.shape[0]
        return jnp.stack(
            [wf_t.reshape(kdim, 4, HID), wb_t.reshape(kdim, 4, HID)],
            axis=2).reshape(kdim, 8 * HID)

    wih = gate_major(W_ih_f.T, W_ih_b.T)
    bih = jnp.stack(
        [(b_ih_f + b_hh_f).reshape(4, HID), (b_ih_b + b_hh_b).reshape(4, HID)],
        axis=1).reshape(1, 8 * HID)
    z64 = jnp.zeros((HID, 4, HID), _f32)
    top = jnp.stack([W_hh_f.T.reshape(HID, 4, HID), z64], axis=2)
    bot = jnp.stack([z64, W_hh_b.T.reshape(HID, 4, HID)], axis=2)
    whh = jnp.concatenate(
        [top.reshape(HID, 8 * HID), bot.reshape(HID, 8 * HID)], axis=0)

    lens2 = lens_i[perm].reshape(B, 1)
    lc2 = learner_count.astype(_f32)[perm].reshape(B, 1)
    rt2 = rating.astype(_f32)[perm].reshape(B, 1)
    du2 = duration.astype(_f32)[perm].reshape(B, 1)
    dp2 = days_since_publish.astype(_f32)[perm].reshape(B, 1)

    out_s = _dense(
        ttl3, k_rows, c_rows, d_rows, tgr, tags_i, lens2,
        lc2, rt2, du2, dp2,
        learner_table, rating_table, duration_table, freshness_table,
        wih, bih, whh,
        W1, b1.reshape(1, -1), g1.reshape(1, -1), be1.reshape(1, -1),
        W2, b2.reshape(1, -1), g2.reshape(1, -1), be2.reshape(1, -1))
    return jnp.zeros((B, ED), _f32).at[perm].set(out_s)
